# baseline (speedup 1.0000x reference)
import functools

import jax
import jax.numpy as jnp
from jax import lax
from jax.experimental import pallas as pl
from jax.experimental.pallas import tpu as pltpu
from jax.experimental.pallas import tpu_sc as plsc

_HASH_SEEDS = (2654435769, 3210233709, 2496678331, 3249880090)
_TBL = 8192
_D = 64
_L = 16
_NC = 2
_NS = 16
_NW = _NC * _NS

_B = 4096
_S = 50
_NB = _S - 1
_RPW = _B // _NW
_TPW = _RPW * _S
_GPC = _RPW // _L


def _gf_hash(x, seed):
    x = x ^ jnp.uint32(seed)
    x = (x ^ (x >> jnp.uint32(16))) * jnp.uint32(2146121005)
    x = (x ^ (x >> jnp.uint32(15))) * jnp.uint32(2221713035)
    x = x ^ (x >> jnp.uint32(16))
    return (x & jnp.uint32(_TBL - 1)).astype(jnp.int32)


def _body(tok_hbm, t0, t1, t2, t3, out_hbm,
          tok_v, idx0, idx1, idx2, idx3,
          a0, a1, a2, a3, b0, b1, b2, b3, oa, ob,
          sga, sgb, soa, sob):
    wid = lax.axis_index("s") * _NC + lax.axis_index("c")

    pltpu.sync_copy(tok_hbm.at[pl.ds(wid * _TPW, _TPW)], tok_v)

    lane = lax.iota(jnp.int32, _L)
    lane50 = lane * jnp.int32(_S)
    idx_refs = (idx0, idx1, idx2, idx3)
    tables = (t0, t1, t2, t3)

    def build_idx(j):
        @plsc.parallel_loop(0, _GPC, unroll=2)
        def _hash_loop(g):
            pos = lane50 + (g * (_L * _S) + j)
            left = plsc.load_gather(tok_v, [pos]).astype(jnp.uint32)
            right = plsc.load_gather(tok_v, [pos + 1]).astype(jnp.uint32)
            bg = (left << jnp.uint32(10)) | right
            base = (j * _GPC + g) * _L
            for h in range(4):
                idx_refs[h][pl.ds(base, _L)] = _gf_hash(bg, _HASH_SEEDS[h])

    set_a = ((a0, a1, a2, a3), oa, sga, soa)
    set_b = ((b0, b1, b2, b3), ob, sgb, sob)

    def issue(j, bset):
        bufs, _, sg, _ = bset
        for h in range(4):
            pltpu.async_copy(
                tables[h].at[idx_refs[h].at[pl.ds(j * _RPW, _RPW)]],
                bufs[h], sg)

    def wait_gathers(bset):
        bufs, _, sg, _ = bset
        for h in range(4):
            pltpu.make_async_copy(
                tables[h].at[idx_refs[h].at[pl.ds(0, _RPW)]],
                bufs[h], sg).wait()

    def wait_out(bset):
        _, o, _, so = bset
        pltpu.make_async_copy(o, out_hbm.at[0, :, 0], so).wait()

    quarter_bf = jnp.full((2 * _L,), 0.25, jnp.bfloat16)

    def combine(bset):
        bufs, o, _, _ = bset
        c0, c1, c2, c3 = bufs

        @plsc.parallel_loop(0, _L, unroll=1)
        def _skew(j):
            colskew = (lane + j) & jnp.int32(_L - 1)
            dvecs = []
            for dt in range(_D // (2 * _L)):
                de = (colskew + jnp.int32(dt * _L)) * jnp.int32(2)
                do = de + jnp.int32(1)
                dvecs.append((de >> jnp.int32(3), de & jnp.int32(7),
                              do >> jnp.int32(3), do & jnp.int32(7)))

            @plsc.parallel_loop(0, _RPW // _L, unroll=1)
            def _bt(bt):
                rowv = lane + bt * _L
                for dt in range(_D // (2 * _L)):
                    colv = colskew + jnp.int32(dt * _L)
                    s = ((plsc.bitcast(plsc.load_gather(c0, [rowv, colv]),
                                       jnp.bfloat16)
                          + plsc.bitcast(plsc.load_gather(c1, [rowv, colv]),
                                         jnp.bfloat16))
                         + (plsc.bitcast(plsc.load_gather(c2, [rowv, colv]),
                                         jnp.bfloat16)
                            + plsc.bitcast(plsc.load_gather(c3, [rowv, colv]),
                                           jnp.bfloat16))) * quarter_bf
                    we, wo = plsc.unpack(s, format=plsc.PackFormat.INTERLEAVED)
                    dhe, dle, dho, dlo = dvecs[dt]
                    plsc.store_scatter(o, [dhe, dle, rowv], we)
                    plsc.store_scatter(o, [dho, dlo, rowv], wo)

    def out_dma(j, bset):
        _, o, _, so = bset
        pltpu.async_copy(o, out_hbm.at[j, :, wid], so)

    build_idx(jnp.int32(0))
    build_idx(jnp.int32(1))
    issue(0, set_a)

    def pair_body(k, carry):
        ja = 2 * k
        jb = ja + 1
        issue(jb, set_b)
        build_idx(jnp.minimum(ja + 2, _NB - 1))
        wait_gathers(set_a)

        @pl.when(k > 0)
        def _():
            wait_out(set_a)

        combine(set_a)
        out_dma(ja, set_a)
        issue(jnp.minimum(ja + 2, _NB - 1), set_a)
        build_idx(jnp.minimum(jb + 2, _NB - 1))
        wait_gathers(set_b)

        @pl.when(k > 0)
        def _():
            wait_out(set_b)

        combine(set_b)
        out_dma(jb, set_b)
        return carry

    lax.fori_loop(0, _NB // 2, pair_body, 0)

    wait_gathers(set_a)
    wait_out(set_a)
    combine(set_a)
    out_dma(_NB - 1, set_a)
    wait_out(set_a)
    wait_out(set_b)


_sc_call = functools.partial(
    pl.kernel,
    out_type=jax.ShapeDtypeStruct((_NB, _D // 8, _NW, 8, _RPW), jnp.float32),
    mesh=plsc.VectorSubcoreMesh(
        core_axis_name="c", subcore_axis_name="s",
        num_cores=_NC, num_subcores=_NS),
    scratch_types=[
        pltpu.VMEM((_TPW,), jnp.int32),
        pltpu.VMEM((_NB * _RPW,), jnp.int32),
        pltpu.VMEM((_NB * _RPW,), jnp.int32),
        pltpu.VMEM((_NB * _RPW,), jnp.int32),
        pltpu.VMEM((_NB * _RPW,), jnp.int32),
        pltpu.VMEM((_RPW, _D // 2), jnp.int32),
        pltpu.VMEM((_RPW, _D // 2), jnp.int32),
        pltpu.VMEM((_RPW, _D // 2), jnp.int32),
        pltpu.VMEM((_RPW, _D // 2), jnp.int32),
        pltpu.VMEM((_RPW, _D // 2), jnp.int32),
        pltpu.VMEM((_RPW, _D // 2), jnp.int32),
        pltpu.VMEM((_RPW, _D // 2), jnp.int32),
        pltpu.VMEM((_RPW, _D // 2), jnp.int32),
        pltpu.VMEM((_D // 8, 8, _RPW), jnp.float32),
        pltpu.VMEM((_D // 8, 8, _RPW), jnp.float32),
        pltpu.SemaphoreType.DMA,
        pltpu.SemaphoreType.DMA,
        pltpu.SemaphoreType.DMA,
        pltpu.SemaphoreType.DMA,
    ],
    compiler_params=pltpu.CompilerParams(
        needs_layout_passes=False, use_tc_tiling_on_sc=False),
)(_body)


def _pack_table(t):
    ti = lax.bitcast_convert_type(
        t.astype(jnp.bfloat16), jnp.uint16).astype(jnp.uint32)
    return (ti[:, 0::2] | (ti[:, 1::2] << jnp.uint32(16))).astype(jnp.int32)


@jax.jit
def kernel(token_ids, table0, table1, table2, table3):
    out = _sc_call(token_ids.reshape(-1),
                   _pack_table(table0), _pack_table(table1),
                   _pack_table(table2), _pack_table(table3))
    return jnp.transpose(out, (2, 4, 0, 1, 3)).reshape(_B, _NB, _D)


# speedup vs baseline: 3.6680x; 3.6680x over previous
import functools

import jax
import jax.numpy as jnp
from jax import lax
from jax.experimental import pallas as pl
from jax.experimental.pallas import tpu as pltpu
from jax.experimental.pallas import tpu_sc as plsc

_HASH_SEEDS = (2654435769, 3210233709, 2496678331, 3249880090)
_TBL = 8192
_D = 64
_L = 16
_NC = 2
_NS = 16
_NW = _NC * _NS

_B = 4096
_S = 50
_NB = _S - 1
_RPW = _B // _NW
_TPW = _RPW * _S
_GPC = _RPW // _L


def _gf_hash(x, seed):
    x = x ^ jnp.uint32(seed)
    x = (x ^ (x >> jnp.uint32(16))) * jnp.uint32(2146121005)
    x = (x ^ (x >> jnp.uint32(15))) * jnp.uint32(2221713035)
    x = x ^ (x >> jnp.uint32(16))
    return (x & jnp.uint32(_TBL - 1)).astype(jnp.int32)


def _body(tok_hbm, t0, t1, t2, t3, out_hbm,
          tok_v, idx0, idx1, idx2, idx3,
          a0, a1, a2, a3, b0, b1, b2, b3, c0_, c1_, c2_, c3_, oa, ob, oc,
          sga, sgb, sgc, soa, sob, soc):
    wid = lax.axis_index("s") * _NC + lax.axis_index("c")

    pltpu.sync_copy(tok_hbm.at[pl.ds(wid * _TPW, _TPW)], tok_v)

    lane = lax.iota(jnp.int32, _L)
    lane50 = lane * jnp.int32(_S)
    idx_refs = (idx0, idx1, idx2, idx3)
    tables = (t0, t1, t2, t3)

    def build_idx(j):
        @plsc.parallel_loop(0, _GPC, unroll=2)
        def _hash_loop(g):
            pos = lane50 + (g * (_L * _S) + j)
            left = plsc.load_gather(tok_v, [pos]).astype(jnp.uint32)
            right = plsc.load_gather(tok_v, [pos + 1]).astype(jnp.uint32)
            bg = (left << jnp.uint32(10)) | right
            base = (j * _GPC + g) * _L
            for h in range(4):
                idx_refs[h][pl.ds(base, _L)] = _gf_hash(bg, _HASH_SEEDS[h])

    sets = (((a0, a1, a2, a3), oa, sga, soa),
            ((b0, b1, b2, b3), ob, sgb, sob),
            ((c0_, c1_, c2_, c3_), oc, sgc, soc))

    def issue(j, bset):
        bufs, _, sg, _ = bset
        for h in range(4):
            pltpu.async_copy(
                tables[h].at[idx_refs[h].at[pl.ds(j * _RPW, _RPW)]],
                bufs[h], sg)

    def wait_gathers(bset):
        bufs, _, sg, _ = bset
        for h in range(4):
            pltpu.make_async_copy(
                tables[h].at[idx_refs[h].at[pl.ds(0, _RPW)]],
                bufs[h], sg).wait()

    def wait_out(bset):
        _, o, _, so = bset
        pltpu.make_async_copy(o, out_hbm.at[0, :, 0], so).wait()

    quarter_bf = jnp.full((2 * _L,), 0.25, jnp.bfloat16)

    def combine(bset):
        bufs, o, _, _ = bset
        c0, c1, c2, c3 = bufs

        @plsc.parallel_loop(0, _L, unroll=1)
        def _skew(j):
            colskew = (lane + j) & jnp.int32(_L - 1)
            dvecs = []
            for dt in range(_D // (2 * _L)):
                de = (colskew + jnp.int32(dt * _L)) * jnp.int32(2)
                do = de + jnp.int32(1)
                dvecs.append((de >> jnp.int32(3), de & jnp.int32(7),
                              do >> jnp.int32(3), do & jnp.int32(7)))

            @plsc.parallel_loop(0, _RPW // _L, unroll=1)
            def _bt(bt):
                rowv = lane + bt * _L
                for dt in range(_D // (2 * _L)):
                    colv = colskew + jnp.int32(dt * _L)
                    s = ((plsc.bitcast(plsc.load_gather(c0, [rowv, colv]),
                                       jnp.bfloat16)
                          + plsc.bitcast(plsc.load_gather(c1, [rowv, colv]),
                                         jnp.bfloat16))
                         + (plsc.bitcast(plsc.load_gather(c2, [rowv, colv]),
                                         jnp.bfloat16)
                            + plsc.bitcast(plsc.load_gather(c3, [rowv, colv]),
                                           jnp.bfloat16))) * quarter_bf
                    we, wo = plsc.unpack(s, format=plsc.PackFormat.INTERLEAVED)
                    dhe, dle, dho, dlo = dvecs[dt]
                    plsc.store_scatter(o, [dhe, dle, rowv], we)
                    plsc.store_scatter(o, [dho, dlo, rowv], wo)

    def out_dma(j, bset):
        _, o, _, so = bset
        pltpu.async_copy(o, out_hbm.at[j, :, wid], so)

    build_idx(jnp.int32(0))
    build_idx(jnp.int32(1))
    issue(0, sets[0])
    issue(1, sets[1])

    def triple_body(k, carry):
        for i in range(3):
            c = 3 * k + i
            nxt = jnp.minimum(c + 2, _NB - 1)
            build_idx(nxt)
            issue(nxt, sets[(i + 2) % 3])
            st = sets[i]
            wait_gathers(st)

            @pl.when(k > 0)
            def _():
                wait_out(st)

            combine(st)
            out_dma(c, st)
        return carry

    lax.fori_loop(0, (_NB - 1) // 3, triple_body, 0)

    wait_gathers(sets[0])
    wait_out(sets[0])
    combine(sets[0])
    out_dma(_NB - 1, sets[0])
    wait_gathers(sets[1])
    wait_out(sets[0])
    wait_out(sets[1])
    wait_out(sets[2])


_sc_call = functools.partial(
    pl.kernel,
    out_type=jax.ShapeDtypeStruct((_NB, _D // 8, _NW, 8, _RPW), jnp.float32),
    mesh=plsc.VectorSubcoreMesh(
        core_axis_name="c", subcore_axis_name="s",
        num_cores=_NC, num_subcores=_NS),
    scratch_types=[
        pltpu.VMEM((_TPW,), jnp.int32),
        pltpu.VMEM((_NB * _RPW,), jnp.int32),
        pltpu.VMEM((_NB * _RPW,), jnp.int32),
        pltpu.VMEM((_NB * _RPW,), jnp.int32),
        pltpu.VMEM((_NB * _RPW,), jnp.int32),
        pltpu.VMEM((_RPW, _D // 2), jnp.int32),
        pltpu.VMEM((_RPW, _D // 2), jnp.int32),
        pltpu.VMEM((_RPW, _D // 2), jnp.int32),
        pltpu.VMEM((_RPW, _D // 2), jnp.int32),
        pltpu.VMEM((_RPW, _D // 2), jnp.int32),
        pltpu.VMEM((_RPW, _D // 2), jnp.int32),
        pltpu.VMEM((_RPW, _D // 2), jnp.int32),
        pltpu.VMEM((_RPW, _D // 2), jnp.int32),
        pltpu.VMEM((_RPW, _D // 2), jnp.int32),
        pltpu.VMEM((_RPW, _D // 2), jnp.int32),
        pltpu.VMEM((_RPW, _D // 2), jnp.int32),
        pltpu.VMEM((_RPW, _D // 2), jnp.int32),
        pltpu.VMEM((_D // 8, 8, _RPW), jnp.float32),
        pltpu.VMEM((_D // 8, 8, _RPW), jnp.float32),
        pltpu.VMEM((_D // 8, 8, _RPW), jnp.float32),
        pltpu.SemaphoreType.DMA,
        pltpu.SemaphoreType.DMA,
        pltpu.SemaphoreType.DMA,
        pltpu.SemaphoreType.DMA,
        pltpu.SemaphoreType.DMA,
        pltpu.SemaphoreType.DMA,
    ],
    compiler_params=pltpu.CompilerParams(
        needs_layout_passes=False, use_tc_tiling_on_sc=False),
)(_body)


def _pack_table(t):
    return lax.bitcast_convert_type(
        t.astype(jnp.bfloat16).reshape(_TBL, _D // 2, 2), jnp.int32)


@jax.jit
def kernel(token_ids, table0, table1, table2, table3):
    out = _sc_call(token_ids.reshape(-1),
                   _pack_table(table0), _pack_table(table1),
                   _pack_table(table2), _pack_table(table3))
    return jnp.transpose(out, (2, 4, 0, 1, 3)).reshape(_B, _NB, _D)
